# baseline (speedup 1.0000x reference)
import functools

import jax
import jax.numpy as jnp
from jax import lax
from jax.experimental import pallas as pl
from jax.experimental.pallas import tpu as pltpu
from jax.experimental.pallas import tpu_sc as plsc

N_FIELDS = 26
VOCAB = 1_000_000
BATCH = 16384

NC = 2
NS = 16
LANES = 16
NW = NC * NS
R = BATCH // NW
SEG = R // 128
NROW = N_FIELDS * SEG
XROWS = BATCH // 128

_mesh = plsc.VectorSubcoreMesh(core_axis_name="c", subcore_axis_name="s")


@functools.partial(
    pl.kernel,
    out_type=jax.ShapeDtypeStruct((BATCH,), jnp.float32),
    mesh=_mesh,
    scratch_types=[
        pltpu.VMEM((NROW, 128), jnp.int32),
        pltpu.VMEM((NROW, 128), jnp.float32),
        pltpu.VMEM((R,), jnp.float32),
        pltpu.SemaphoreType.DMA,
        pltpu.SemaphoreType.DMA,
    ],
)
def _lr_kernel(xt_hbm, tab_hbm, out_hbm, idx_v, gat_v, out_v, sem_x, sem_g):
    wid = lax.axis_index("s") * NC + lax.axis_index("c")
    def x_copy(f):
        return pltpu.make_async_copy(
            xt_hbm.at[pl.ds(f * (BATCH // 128) + wid * SEG, SEG), :],
            idx_v.at[pl.ds(f * SEG, SEG), :],
            sem_x,
        )
    for f in range(N_FIELDS):
        x_copy(f).start()
    for f in range(N_FIELDS):
        x_copy(f).wait()

    def off_body(k, carry):
        off = (k // SEG) * VOCAB
        for c in range(128 // LANES):
            sl = (k, pl.ds(c * LANES, LANES))
            idx_v[sl] = idx_v[sl] + off
        return carry
    lax.fori_loop(0, NROW, off_body, 0)

    def g_copy(k):
        return pltpu.make_async_copy(tab_hbm.at[idx_v.at[k]], gat_v.at[k], sem_g)
    def fire_body(k, carry):
        g_copy(k).start()
        return carry
    lax.fori_loop(0, NROW, fire_body, 0)
    def drain_body(k, carry):
        g_copy(k).wait()
        return carry
    lax.fori_loop(0, NROW, drain_body, 0)

    def red_body(j, carry):
        s = j // (128 // LANES)
        c = (j % (128 // LANES)) * LANES
        acc = gat_v[s, pl.ds(c, LANES)]
        for f in range(1, N_FIELDS):
            acc = acc + gat_v[f * SEG + s, pl.ds(c, LANES)]
        out_v[pl.ds(j * LANES, LANES)] = acc
        return carry
    lax.fori_loop(0, R // LANES, red_body, 0)

    pltpu.sync_copy(out_v, out_hbm.at[pl.ds(wid * R, R)])


def kernel(x, tables):
    xt = jnp.transpose(x.astype(jnp.int32)).reshape(N_FIELDS * XROWS, 128)
    tab = tables.reshape(N_FIELDS * VOCAB)
    return _lr_kernel(xt, tab)
